# baseline (speedup 1.0000x reference)
import functools

import jax
import jax.numpy as jnp
from jax import lax
from jax.experimental import pallas as pl
from jax.experimental.pallas import tpu as pltpu
from jax.experimental.pallas import tpu_sc as plsc

_EPS = 1e-5
_N = 100000
_G = 64
_CHUNK = 2000
_NW = 32
_SEG1 = 6256
_SEG1_LAST = _N - 15 * _SEG1
_SEG2 = 6256
_SEG2_LAST = _N - 15 * _SEG2


def _sc1_body(nchunks, x_hbm, src_hbm, dst_hbm, ew_hbm,
              out0_hbm, out1_hbm,
              s_sp, src_a, src_b, dst_a, dst_b, ew_a, ew_b,
              xv_a, xv_b, msg_a, msg_b, sem_l, sem_g, sem_s):
    cid = lax.axis_index("c")
    sid = lax.axis_index("s")
    wid = cid * 16 + sid
    base1 = sid * _SEG1

    def z1(j, c):
        xv_a[pl.ds(j * 16, 16)] = jnp.zeros((16,), jnp.float32)
        return c

    lax.fori_loop(0, _CHUNK // 16, z1, 0)
    for off in range(0, 3 * _CHUNK, _CHUNK):
        pltpu.sync_copy(xv_a, s_sp.at[pl.ds(base1 + off, _CHUNK)])

    @pl.when(sid < 15)
    def _():
        pltpu.sync_copy(xv_a.at[pl.ds(0, _SEG1 - 3 * _CHUNK)],
                        s_sp.at[pl.ds(base1 + 3 * _CHUNK,
                                      _SEG1 - 3 * _CHUNK)])

    @pl.when(sid == 15)
    def _():
        pltpu.sync_copy(xv_a.at[pl.ds(0, _SEG1_LAST - 3 * _CHUNK)],
                        s_sp.at[pl.ds(base1 + 3 * _CHUNK,
                                      _SEG1_LAST - 3 * _CHUNK)])

    plsc.subcore_barrier()

    row0 = wid * (nchunks * _CHUNK)
    bufs = ((src_a, dst_a, ew_a, xv_a, msg_a),
            (src_b, dst_b, ew_b, xv_b, msg_b))

    def fire_loads(kc, src_v, dst_v, ew_v):
        e0 = row0 + kc * _CHUNK
        pltpu.async_copy(src_hbm.at[pl.ds(e0, _CHUNK)], src_v, sem_l)
        pltpu.async_copy(dst_hbm.at[pl.ds(e0, _CHUNK)], dst_v, sem_l)
        pltpu.async_copy(ew_hbm.at[pl.ds(e0, _CHUNK)], ew_v, sem_l)

    def drain_loads(src_v, dst_v, ew_v):
        pltpu.make_async_copy(src_hbm.at[pl.ds(0, _CHUNK)], src_v,
                              sem_l).wait()
        pltpu.make_async_copy(dst_hbm.at[pl.ds(0, _CHUNK)], dst_v,
                              sem_l).wait()
        pltpu.make_async_copy(ew_hbm.at[pl.ds(0, _CHUNK)], ew_v,
                              sem_l).wait()

    def fire_gathers(src_v, xv):
        pltpu.async_copy(x_hbm.at[src_v], xv, sem_g)

    fire_loads(0, src_a, dst_a, ew_a)
    drain_loads(src_a, dst_a, ew_a)
    fire_gathers(src_a, xv_a)

    def step(kc, cur, nxt):
        src_c, dst_c, ew_c, xv_c, msg_c = cur
        src_n, dst_n, ew_n, xv_n, msg_n = nxt

        @pl.when(kc >= 1)
        def _():
            pltpu.make_async_copy(x_hbm.at[pl.ds(0, _CHUNK)], msg_n,
                                  sem_s).wait()

        @pl.when(kc + 1 < nchunks)
        def _():
            fire_loads(kc + 1, src_n, dst_n, ew_n)

        pltpu.make_async_copy(x_hbm.at[pl.ds(0, _CHUNK)], xv_c,
                              sem_g).wait()

        def mul(j, c):
            sl = pl.ds(j * 16, 16)
            msg_c[sl] = xv_c[sl] * ew_c[sl]
            return c

        lax.fori_loop(0, _CHUNK // 16, mul, 0)
        pltpu.async_copy(msg_c, s_sp.at[dst_c], sem_s, add=True)

        @pl.when(kc + 1 < nchunks)
        def _():
            drain_loads(src_n, dst_n, ew_n)
            fire_gathers(src_n, xv_n)

    def pair(kp, carry):
        step(2 * kp, bufs[0], bufs[1])
        step(2 * kp + 1, bufs[1], bufs[0])
        return carry

    lax.fori_loop(0, nchunks // 2, pair, 0)
    pltpu.make_async_copy(x_hbm.at[pl.ds(0, _CHUNK)], msg_a, sem_s).wait()
    plsc.subcore_barrier()

    def wout(off, size):
        pltpu.sync_copy(s_sp.at[pl.ds(base1 + off, size)],
                        xv_a.at[pl.ds(0, size)])

        @pl.when(cid == 0)
        def _():
            pltpu.sync_copy(xv_a.at[pl.ds(0, size)],
                            out0_hbm.at[pl.ds(base1 + off, size)])

        @pl.when(cid == 1)
        def _():
            pltpu.sync_copy(xv_a.at[pl.ds(0, size)],
                            out1_hbm.at[pl.ds(base1 + off, size)])

    for off in range(0, 3 * _CHUNK, _CHUNK):
        wout(off, _CHUNK)

    @pl.when(sid < 15)
    def _():
        wout(3 * _CHUNK, _SEG1 - 3 * _CHUNK)

    @pl.when(sid == 15)
    def _():
        wout(3 * _CHUNK, _SEG1_LAST - 3 * _CHUNK)


def _sc_pass1(xf, src2, dst2, ew2, nchunks):
    mesh = plsc.VectorSubcoreMesh(core_axis_name="c", subcore_axis_name="s")
    fn = functools.partial(
        pl.kernel,
        mesh=mesh,
        compiler_params=pltpu.CompilerParams(
            use_tc_tiling_on_sc=False, needs_layout_passes=False),
        out_type=[jax.ShapeDtypeStruct((_N,), jnp.float32)] * 2,
        scratch_types=[
            pltpu.VMEM_SHARED((_N,), jnp.float32),
            pltpu.VMEM((_CHUNK,), jnp.int32),
            pltpu.VMEM((_CHUNK,), jnp.int32),
            pltpu.VMEM((_CHUNK,), jnp.int32),
            pltpu.VMEM((_CHUNK,), jnp.int32),
            pltpu.VMEM((_CHUNK,), jnp.float32),
            pltpu.VMEM((_CHUNK,), jnp.float32),
            pltpu.VMEM((_CHUNK,), jnp.float32),
            pltpu.VMEM((_CHUNK,), jnp.float32),
            pltpu.VMEM((_CHUNK,), jnp.float32),
            pltpu.VMEM((_CHUNK,), jnp.float32),
            pltpu.SemaphoreType.DMA,
            pltpu.SemaphoreType.DMA,
            pltpu.SemaphoreType.DMA,
        ],
    )(functools.partial(_sc1_body, nchunks))
    return fn(xf, src2, dst2, ew2)


def _sc2_body(nchunks, hn_hbm, src_hbm, dst_hbm, ew_hbm,
              out0_hbm, out1_hbm,
              ag_sp, src_v, dst_v_a, dst_v_b, ew_v, rows_a, rows_b, tbuf,
              sem_l, sem_g, sem_s):
    cid = lax.axis_index("c")
    sid = lax.axis_index("s")
    wid = cid * 16 + sid
    row_base = sid * _SEG2

    half = (lax.iota(jnp.int32, 16) >> 3) & 1
    col = lax.iota(jnp.int32, 16) & 7

    def zrow(j, e):
        plsc.store_scatter(rows_a, [e, col], jnp.zeros((16,), jnp.float32))
        return e + 2

    lax.fori_loop(0, _CHUNK * 8 // 16, zrow, half)
    for off in range(0, 3 * _CHUNK, _CHUNK):
        pltpu.sync_copy(rows_a, ag_sp.at[pl.ds(row_base + off, _CHUNK), :])

    @pl.when(sid < 15)
    def _():
        pltpu.sync_copy(rows_a.at[pl.ds(0, _SEG2 - 3 * _CHUNK), :],
                        ag_sp.at[pl.ds(row_base + 3 * _CHUNK,
                                       _SEG2 - 3 * _CHUNK), :])

    @pl.when(sid == 15)
    def _():
        pltpu.sync_copy(rows_a.at[pl.ds(0, _SEG2_LAST - 3 * _CHUNK), :],
                        ag_sp.at[pl.ds(row_base + 3 * _CHUNK,
                                       _SEG2_LAST - 3 * _CHUNK), :])

    plsc.subcore_barrier()

    row0 = wid * (nchunks * _CHUNK)

    def do_chunk(kc, rows_v, dst_v):
        @pl.when(kc >= 2)
        def _():
            pltpu.make_async_copy(hn_hbm.at[pl.ds(0, _CHUNK), :], rows_v,
                                  sem_s).wait()

        e0 = row0 + kc * _CHUNK
        l1 = pltpu.async_copy(src_hbm.at[pl.ds(e0, _CHUNK)], src_v, sem_l)
        l2 = pltpu.async_copy(dst_hbm.at[pl.ds(e0, _CHUNK)], dst_v, sem_l)
        l3 = pltpu.async_copy(ew_hbm.at[pl.ds(e0, _CHUNK)], ew_v, sem_l)
        l1.wait()
        l2.wait()
        l3.wait()
        pltpu.async_copy(hn_hbm.at[src_v], rows_v, sem_g).wait()

        def mul(j, e):
            w = plsc.load_gather(ew_v, [e])
            v = plsc.load_gather(rows_v, [e, col])
            plsc.store_scatter(rows_v, [e, col], v * w)
            return e + 2

        lax.fori_loop(0, _CHUNK * 8 // 16, mul, half)
        pltpu.async_copy(rows_v, ag_sp.at[dst_v], sem_s, add=True)

    def pair(kp, carry):
        do_chunk(2 * kp, rows_a, dst_v_a)
        do_chunk(2 * kp + 1, rows_b, dst_v_b)
        return carry

    lax.fori_loop(0, nchunks // 2, pair, 0)
    pltpu.make_async_copy(hn_hbm.at[pl.ds(0, _CHUNK), :], rows_a,
                          sem_s).wait()
    pltpu.make_async_copy(hn_hbm.at[pl.ds(0, _CHUNK), :], rows_b,
                          sem_s).wait()
    plsc.subcore_barrier()

    ri = lax.iota(jnp.int32, 16)

    def wout(off, size):
        pltpu.sync_copy(ag_sp.at[pl.ds(row_base + off, size), :],
                        rows_a.at[pl.ds(0, size), :])
        for f in range(8):
            fv = jnp.full((16,), f, jnp.int32)

            def tr(j, c):
                tbuf[pl.ds(j * 16, 16)] = plsc.load_gather(
                    rows_a, [ri + j * 16, fv])
                return c

            lax.fori_loop(0, size // 16, tr, 0)

            @pl.when(cid == 0)
            def _():
                pltpu.sync_copy(tbuf.at[pl.ds(0, size)],
                                out0_hbm.at[f, pl.ds(row_base + off, size)])

            @pl.when(cid == 1)
            def _():
                pltpu.sync_copy(tbuf.at[pl.ds(0, size)],
                                out1_hbm.at[f, pl.ds(row_base + off, size)])

    for off in range(0, 3 * _CHUNK, _CHUNK):
        wout(off, _CHUNK)

    @pl.when(sid < 15)
    def _():
        wout(3 * _CHUNK, _SEG2 - 3 * _CHUNK)

    @pl.when(sid == 15)
    def _():
        wout(3 * _CHUNK, _SEG2_LAST - 3 * _CHUNK)


def _sc_pass2(hn, src2, dst2, ew2, nchunks):
    mesh = plsc.VectorSubcoreMesh(core_axis_name="c", subcore_axis_name="s")
    fn = functools.partial(
        pl.kernel,
        mesh=mesh,
        compiler_params=pltpu.CompilerParams(
            use_tc_tiling_on_sc=False, needs_layout_passes=False),
        out_type=[jax.ShapeDtypeStruct((8, _N), jnp.float32)] * 2,
        scratch_types=[
            pltpu.VMEM_SHARED((_N, 8), jnp.float32),
            pltpu.VMEM((_CHUNK,), jnp.int32),
            pltpu.VMEM((_CHUNK,), jnp.int32),
            pltpu.VMEM((_CHUNK,), jnp.int32),
            pltpu.VMEM((_CHUNK,), jnp.float32),
            pltpu.VMEM((_CHUNK, 8), jnp.float32),
            pltpu.VMEM((_CHUNK, 8), jnp.float32),
            pltpu.VMEM((_CHUNK,), jnp.float32),
            pltpu.SemaphoreType.DMA,
            pltpu.SemaphoreType.DMA,
            pltpu.SemaphoreType.DMA,
        ],
    )(functools.partial(_sc2_body, nchunks))
    return fn(hn, src2, dst2, ew2)


_BN = 4096
_HP = lax.Precision.HIGHEST


def _mask_of(b_row, i):
    giota = lax.broadcasted_iota(jnp.int32, (_G, 1), 0)
    node = i * _BN + lax.broadcasted_iota(jnp.int32, (1, _BN), 1)
    return jnp.where((b_row == giota) & (node < _N), 1.0, 0.0)


def _ab_body(x_ref, s0_ref, s1_ref, b_ref, w1_ref, r1_ref, b1_ref,
             gnw_ref, gnb_ref, gnms_ref,
             hnt_ref, hn_ref, sums_ref, sumsq_ref, cnt_ref):
    p = pl.program_id(0)
    i = pl.program_id(1)
    x = x_ref[...]
    s = s0_ref[...] + s1_ref[...]
    h = jnp.maximum(w1_ref[...] * s + r1_ref[...] * x + b1_ref[...], 0.0)
    mask = _mask_of(b_ref[...], i)
    dnl = (((1,), (1,)), ((), ()))

    @pl.when(p == 0)
    def _():
        @pl.when(i == 0)
        def _():
            sums_ref[...] = jnp.zeros_like(sums_ref)
            sumsq_ref[...] = jnp.zeros_like(sumsq_ref)
            cnt_ref[...] = jnp.zeros_like(cnt_ref)

        sums_ref[...] += lax.dot_general(
            h, mask, dnl, preferred_element_type=jnp.float32, precision=_HP)
        sumsq_ref[...] += lax.dot_general(
            h * h, mask, dnl, preferred_element_type=jnp.float32,
            precision=_HP)
        cnt_ref[...] += lax.dot_general(
            jnp.ones_like(h), mask, dnl, preferred_element_type=jnp.float32,
            precision=_HP)

    @pl.when(p == 1)
    def _():
        cnt = cnt_ref[...]
        mean = sums_ref[...] / cnt
        ex2 = sumsq_ref[...] / cnt
        mm = mean * gnms_ref[...]
        var = ex2 - 2.0 * mean * mm + mm * mm
        istd = lax.rsqrt(var + _EPS)
        ag = gnw_ref[...] * istd
        bg = gnb_ref[...] - gnw_ref[...] * mm * istd
        dns = (((1,), (0,)), ((), ()))
        an = lax.dot_general(ag, mask, dns,
                             preferred_element_type=jnp.float32,
                             precision=_HP)
        bn_ = lax.dot_general(bg, mask, dns,
                              preferred_element_type=jnp.float32,
                              precision=_HP)
        hnt = an * h + bn_
        hnt_ref[...] = hnt
        hn_ref[...] = hnt.T


def _tc_ab(x1, s0r, s1r, br, w1c, r1c, b1c, gnwc, gnbc, gnmsc):
    grid = (_N + _BN - 1) // _BN
    row = pl.BlockSpec((1, _BN), lambda p, i: (0, i))
    w = pl.BlockSpec((8, 1), lambda p, i: (0, 0))
    st = pl.BlockSpec((8, _G), lambda p, i: (0, 0))
    return pl.pallas_call(
        _ab_body,
        grid=(2, grid),
        in_specs=[row, row, row, row, w, w, w, w, w, w],
        out_specs=[pl.BlockSpec((8, _BN), lambda p, i: (0, i)),
                   pl.BlockSpec((_BN, 8), lambda p, i: (i, 0)),
                   st, st, st],
        out_shape=[jax.ShapeDtypeStruct((8, _N), jnp.float32),
                   jax.ShapeDtypeStruct((_N, 8), jnp.float32)] +
                  [jax.ShapeDtypeStruct((8, _G), jnp.float32)] * 3,
    )(x1, s0r, s1r, br, w1c, r1c, b1c, gnwc, gnbc, gnmsc)


def _final_body(hnt_ref, a0_ref, a1_ref, wmu_ref, rmu_ref, bmu_ref,
                wlv_ref, rlv_ref, blv_ref, mu_ref, lv_ref):
    a = a0_ref[...] + a1_ref[...]
    hnt = hnt_ref[...]
    dn = (((1,), (0,)), ((), ()))
    mu_ref[...] = (lax.dot_general(wmu_ref[...], a, dn,
                                   preferred_element_type=jnp.float32,
                                   precision=_HP)
                   + lax.dot_general(rmu_ref[...], hnt, dn,
                                     preferred_element_type=jnp.float32,
                                     precision=_HP)
                   + bmu_ref[...]).T
    lv_ref[...] = (lax.dot_general(wlv_ref[...], a, dn,
                                   preferred_element_type=jnp.float32,
                                   precision=_HP)
                   + lax.dot_general(rlv_ref[...], hnt, dn,
                                     preferred_element_type=jnp.float32,
                                     precision=_HP)
                   + blv_ref[...]).T


def _tc_final(hnt, a0t, a1t, wmut, rmut, bmut, wlvt, rlvt, blvt):
    grid = (_N + _BN - 1) // _BN
    blk8 = pl.BlockSpec((8, _BN), lambda i: (0, i))
    blk16 = pl.BlockSpec((_BN, 16), lambda i: (i, 0))
    w168 = pl.BlockSpec((16, 8), lambda i: (0, 0))
    w161 = pl.BlockSpec((16, 1), lambda i: (0, 0))
    return pl.pallas_call(
        _final_body,
        grid=(grid,),
        in_specs=[blk8, blk8, blk8, w168, w168, w161, w168, w168, w161],
        out_specs=[blk16, blk16],
        out_shape=[jax.ShapeDtypeStruct((_N, 16), jnp.float32)] * 2,
    )(hnt, a0t, a1t, wmut, rmut, bmut, wlvt, rlvt, blvt)


def kernel(x, edge_index, edge_weight, batch, W1, b1, R1,
           gn_w, gn_b, gn_ms, Wmu, bmu, Rmu, Wlv, blv, Rlv):
    e = edge_weight.shape[0]
    nchunks = -(-e // (_NW * _CHUNK))
    nchunks += nchunks % 2
    ep = _NW * _CHUNK * nchunks
    pad = ep - e

    if pad:
        pad_idx = (jnp.arange(pad, dtype=jnp.int32) * 16) % _N
        src2 = jnp.concatenate([edge_index[0], pad_idx])
        dst2 = jnp.concatenate([edge_index[1], pad_idx])
        ew2 = jnp.concatenate([edge_weight, jnp.zeros((pad,), jnp.float32)])
    else:
        src2, dst2, ew2 = edge_index[0], edge_index[1], edge_weight

    xf = x[:, 0]
    s0, s1 = _sc_pass1(xf, src2, dst2, ew2, nchunks)

    hnt, hn, _, _, _ = _tc_ab(
        xf[None, :], s0[None, :], s1[None, :], batch[None, :],
        W1.reshape(8, 1), R1.reshape(8, 1), b1[:, None],
        gn_w[:, None], gn_b[:, None], gn_ms[:, None])

    a0t, a1t = _sc_pass2(hn, src2, dst2, ew2, nchunks)

    mu, lv = _tc_final(hnt, a0t, a1t,
                       Wmu.T, Rmu.T, bmu[:, None],
                       Wlv.T, Rlv.T, blv[:, None])
    return (mu, lv)


# speedup vs baseline: 1.0761x; 1.0761x over previous
import functools

import jax
import jax.numpy as jnp
from jax import lax
from jax.experimental import pallas as pl
from jax.experimental.pallas import tpu as pltpu
from jax.experimental.pallas import tpu_sc as plsc

_EPS = 1e-5
_N = 100000
_G = 64
_CHUNK = 2000
_NW = 32
_SEG1 = 6256
_SEG1_LAST = _N - 15 * _SEG1
_SEG2 = 6256
_SEG2_LAST = _N - 15 * _SEG2


def _sc1_body(nchunks, x_hbm, src_hbm, dst_hbm, ew_hbm,
              out0_hbm, out1_hbm,
              s_sp, src_a, src_b, dst_a, dst_b, ew_a, ew_b,
              xv_a, xv_b, msg_a, msg_b, sem_l, sem_g, sem_s):
    cid = lax.axis_index("c")
    sid = lax.axis_index("s")
    wid = cid * 16 + sid
    base1 = sid * _SEG1

    def z1(j, c):
        xv_a[pl.ds(j * 16, 16)] = jnp.zeros((16,), jnp.float32)
        return c

    lax.fori_loop(0, _CHUNK // 16, z1, 0)
    for off in range(0, 3 * _CHUNK, _CHUNK):
        pltpu.sync_copy(xv_a, s_sp.at[pl.ds(base1 + off, _CHUNK)])

    @pl.when(sid < 15)
    def _():
        pltpu.sync_copy(xv_a.at[pl.ds(0, _SEG1 - 3 * _CHUNK)],
                        s_sp.at[pl.ds(base1 + 3 * _CHUNK,
                                      _SEG1 - 3 * _CHUNK)])

    @pl.when(sid == 15)
    def _():
        pltpu.sync_copy(xv_a.at[pl.ds(0, _SEG1_LAST - 3 * _CHUNK)],
                        s_sp.at[pl.ds(base1 + 3 * _CHUNK,
                                      _SEG1_LAST - 3 * _CHUNK)])

    plsc.subcore_barrier()

    row0 = wid * (nchunks * _CHUNK)
    bufs = ((src_a, dst_a, ew_a, xv_a, msg_a),
            (src_b, dst_b, ew_b, xv_b, msg_b))

    def fire_loads(kc, src_v, dst_v, ew_v):
        e0 = row0 + kc * _CHUNK
        pltpu.async_copy(src_hbm.at[pl.ds(e0, _CHUNK)], src_v, sem_l)
        pltpu.async_copy(dst_hbm.at[pl.ds(e0, _CHUNK)], dst_v, sem_l)
        pltpu.async_copy(ew_hbm.at[pl.ds(e0, _CHUNK)], ew_v, sem_l)

    def drain_loads(src_v, dst_v, ew_v):
        pltpu.make_async_copy(src_hbm.at[pl.ds(0, _CHUNK)], src_v,
                              sem_l).wait()
        pltpu.make_async_copy(dst_hbm.at[pl.ds(0, _CHUNK)], dst_v,
                              sem_l).wait()
        pltpu.make_async_copy(ew_hbm.at[pl.ds(0, _CHUNK)], ew_v,
                              sem_l).wait()

    def fire_gathers(src_v, xv):
        pltpu.async_copy(x_hbm.at[src_v], xv, sem_g)

    fire_loads(0, src_a, dst_a, ew_a)
    drain_loads(src_a, dst_a, ew_a)
    fire_gathers(src_a, xv_a)

    def step(kc, cur, nxt):
        src_c, dst_c, ew_c, xv_c, msg_c = cur
        src_n, dst_n, ew_n, xv_n, msg_n = nxt

        @pl.when(kc >= 1)
        def _():
            pltpu.make_async_copy(x_hbm.at[pl.ds(0, _CHUNK)], msg_n,
                                  sem_s).wait()

        @pl.when(kc + 1 < nchunks)
        def _():
            fire_loads(kc + 1, src_n, dst_n, ew_n)

        pltpu.make_async_copy(x_hbm.at[pl.ds(0, _CHUNK)], xv_c,
                              sem_g).wait()

        def mul(j, c):
            sl = pl.ds(j * 16, 16)
            msg_c[sl] = xv_c[sl] * ew_c[sl]
            return c

        lax.fori_loop(0, _CHUNK // 16, mul, 0)
        pltpu.async_copy(msg_c, s_sp.at[dst_c], sem_s, add=True)

        @pl.when(kc + 1 < nchunks)
        def _():
            drain_loads(src_n, dst_n, ew_n)
            fire_gathers(src_n, xv_n)

    def pair(kp, carry):
        step(2 * kp, bufs[0], bufs[1])
        step(2 * kp + 1, bufs[1], bufs[0])
        return carry

    lax.fori_loop(0, nchunks // 2, pair, 0)
    pltpu.make_async_copy(x_hbm.at[pl.ds(0, _CHUNK)], msg_a, sem_s).wait()
    plsc.subcore_barrier()

    def wout(off, size):
        pltpu.sync_copy(s_sp.at[pl.ds(base1 + off, size)],
                        xv_a.at[pl.ds(0, size)])

        @pl.when(cid == 0)
        def _():
            pltpu.sync_copy(xv_a.at[pl.ds(0, size)],
                            out0_hbm.at[pl.ds(base1 + off, size)])

        @pl.when(cid == 1)
        def _():
            pltpu.sync_copy(xv_a.at[pl.ds(0, size)],
                            out1_hbm.at[pl.ds(base1 + off, size)])

    for off in range(0, 3 * _CHUNK, _CHUNK):
        wout(off, _CHUNK)

    @pl.when(sid < 15)
    def _():
        wout(3 * _CHUNK, _SEG1 - 3 * _CHUNK)

    @pl.when(sid == 15)
    def _():
        wout(3 * _CHUNK, _SEG1_LAST - 3 * _CHUNK)


def _sc_pass1(xf, src2, dst2, ew2, nchunks):
    mesh = plsc.VectorSubcoreMesh(core_axis_name="c", subcore_axis_name="s")
    fn = functools.partial(
        pl.kernel,
        mesh=mesh,
        compiler_params=pltpu.CompilerParams(
            use_tc_tiling_on_sc=False, needs_layout_passes=False),
        out_type=[jax.ShapeDtypeStruct((_N,), jnp.float32)] * 2,
        scratch_types=[
            pltpu.VMEM_SHARED((_N,), jnp.float32),
            pltpu.VMEM((_CHUNK,), jnp.int32),
            pltpu.VMEM((_CHUNK,), jnp.int32),
            pltpu.VMEM((_CHUNK,), jnp.int32),
            pltpu.VMEM((_CHUNK,), jnp.int32),
            pltpu.VMEM((_CHUNK,), jnp.float32),
            pltpu.VMEM((_CHUNK,), jnp.float32),
            pltpu.VMEM((_CHUNK,), jnp.float32),
            pltpu.VMEM((_CHUNK,), jnp.float32),
            pltpu.VMEM((_CHUNK,), jnp.float32),
            pltpu.VMEM((_CHUNK,), jnp.float32),
            pltpu.SemaphoreType.DMA,
            pltpu.SemaphoreType.DMA,
            pltpu.SemaphoreType.DMA,
        ],
    )(functools.partial(_sc1_body, nchunks))
    return fn(xf, src2, dst2, ew2)


def _sc2_body(nchunks, hn_hbm, src_hbm, dst_hbm, ew_hbm,
              out0_hbm, out1_hbm,
              ag_sp, src_v, dst_v_a, dst_v_b, ew_v, rows_a, rows_b, tbuf,
              sem_l, sem_g, sem_s):
    cid = lax.axis_index("c")
    sid = lax.axis_index("s")
    wid = cid * 16 + sid
    row_base = sid * _SEG2

    half = (lax.iota(jnp.int32, 16) >> 3) & 1
    col = lax.iota(jnp.int32, 16) & 7

    def zrow(j, e):
        plsc.store_scatter(rows_a, [e, col], jnp.zeros((16,), jnp.float32))
        return e + 2

    lax.fori_loop(0, _CHUNK * 8 // 16, zrow, half)
    for off in range(0, 3 * _CHUNK, _CHUNK):
        pltpu.sync_copy(rows_a, ag_sp.at[pl.ds(row_base + off, _CHUNK), :])

    @pl.when(sid < 15)
    def _():
        pltpu.sync_copy(rows_a.at[pl.ds(0, _SEG2 - 3 * _CHUNK), :],
                        ag_sp.at[pl.ds(row_base + 3 * _CHUNK,
                                       _SEG2 - 3 * _CHUNK), :])

    @pl.when(sid == 15)
    def _():
        pltpu.sync_copy(rows_a.at[pl.ds(0, _SEG2_LAST - 3 * _CHUNK), :],
                        ag_sp.at[pl.ds(row_base + 3 * _CHUNK,
                                       _SEG2_LAST - 3 * _CHUNK), :])

    plsc.subcore_barrier()

    row0 = wid * (nchunks * _CHUNK)

    def do_chunk(kc, rows_v, dst_v):
        @pl.when(kc >= 2)
        def _():
            pltpu.make_async_copy(hn_hbm.at[pl.ds(0, _CHUNK), :], rows_v,
                                  sem_s).wait()

        e0 = row0 + kc * _CHUNK
        l1 = pltpu.async_copy(src_hbm.at[pl.ds(e0, _CHUNK)], src_v, sem_l)
        l2 = pltpu.async_copy(dst_hbm.at[pl.ds(e0, _CHUNK)], dst_v, sem_l)
        l3 = pltpu.async_copy(ew_hbm.at[pl.ds(e0, _CHUNK)], ew_v, sem_l)
        l1.wait()
        l2.wait()
        l3.wait()
        pltpu.async_copy(hn_hbm.at[src_v], rows_v, sem_g).wait()

        def mul(j, e):
            w = plsc.load_gather(ew_v, [e])
            v = plsc.load_gather(rows_v, [e, col])
            plsc.store_scatter(rows_v, [e, col], v * w)
            return e + 2

        lax.fori_loop(0, _CHUNK * 8 // 16, mul, half)
        pltpu.async_copy(rows_v, ag_sp.at[dst_v], sem_s, add=True)

    def pair(kp, carry):
        do_chunk(2 * kp, rows_a, dst_v_a)
        do_chunk(2 * kp + 1, rows_b, dst_v_b)
        return carry

    lax.fori_loop(0, nchunks // 2, pair, 0)
    pltpu.make_async_copy(hn_hbm.at[pl.ds(0, _CHUNK), :], rows_a,
                          sem_s).wait()
    pltpu.make_async_copy(hn_hbm.at[pl.ds(0, _CHUNK), :], rows_b,
                          sem_s).wait()
    plsc.subcore_barrier()

    ri = lax.iota(jnp.int32, 16)

    def wout(off, size):
        pltpu.sync_copy(ag_sp.at[pl.ds(row_base + off, size), :],
                        rows_a.at[pl.ds(0, size), :])
        for f in range(8):
            fv = jnp.full((16,), f, jnp.int32)

            def tr(j, c):
                tbuf[pl.ds(j * 16, 16)] = plsc.load_gather(
                    rows_a, [ri + j * 16, fv])
                return c

            lax.fori_loop(0, size // 16, tr, 0)

            @pl.when(cid == 0)
            def _():
                pltpu.sync_copy(tbuf.at[pl.ds(0, size)],
                                out0_hbm.at[f, pl.ds(row_base + off, size)])

            @pl.when(cid == 1)
            def _():
                pltpu.sync_copy(tbuf.at[pl.ds(0, size)],
                                out1_hbm.at[f, pl.ds(row_base + off, size)])

    for off in range(0, 3 * _CHUNK, _CHUNK):
        wout(off, _CHUNK)

    @pl.when(sid < 15)
    def _():
        wout(3 * _CHUNK, _SEG2 - 3 * _CHUNK)

    @pl.when(sid == 15)
    def _():
        wout(3 * _CHUNK, _SEG2_LAST - 3 * _CHUNK)


def _sc_pass2(hn, src2, dst2, ew2, nchunks):
    mesh = plsc.VectorSubcoreMesh(core_axis_name="c", subcore_axis_name="s")
    fn = functools.partial(
        pl.kernel,
        mesh=mesh,
        compiler_params=pltpu.CompilerParams(
            use_tc_tiling_on_sc=False, needs_layout_passes=False),
        out_type=[jax.ShapeDtypeStruct((8, _N), jnp.float32)] * 2,
        scratch_types=[
            pltpu.VMEM_SHARED((_N, 8), jnp.float32),
            pltpu.VMEM((_CHUNK,), jnp.int32),
            pltpu.VMEM((_CHUNK,), jnp.int32),
            pltpu.VMEM((_CHUNK,), jnp.int32),
            pltpu.VMEM((_CHUNK,), jnp.float32),
            pltpu.VMEM((_CHUNK, 8), jnp.float32),
            pltpu.VMEM((_CHUNK, 8), jnp.float32),
            pltpu.VMEM((_CHUNK,), jnp.float32),
            pltpu.SemaphoreType.DMA,
            pltpu.SemaphoreType.DMA,
            pltpu.SemaphoreType.DMA,
        ],
    )(functools.partial(_sc2_body, nchunks))
    return fn(hn, src2, dst2, ew2)


_BN = 4096
_HP = lax.Precision.HIGHEST


def _mask_of(b_row, i):
    giota = lax.broadcasted_iota(jnp.int32, (_G, 1), 0)
    node = i * _BN + lax.broadcasted_iota(jnp.int32, (1, _BN), 1)
    return jnp.where((b_row == giota) & (node < _N), 1.0, 0.0)


def _ab_body(x_ref, s0_ref, s1_ref, b_ref, w1_ref, r1_ref, b1_ref,
             gnw_ref, gnb_ref, gnms_ref,
             hnt_ref, hn_ref, sums_ref, sumsq_ref, cnt_ref):
    p = pl.program_id(0)
    i = pl.program_id(1)
    x = x_ref[...]
    s = s0_ref[...] + s1_ref[...]
    h = jnp.maximum(w1_ref[...] * s + r1_ref[...] * x + b1_ref[...], 0.0)
    mask = _mask_of(b_ref[...], i)
    dnl = (((1,), (1,)), ((), ()))

    @pl.when(p == 0)
    def _():
        @pl.when(i == 0)
        def _():
            sums_ref[...] = jnp.zeros_like(sums_ref)
            sumsq_ref[...] = jnp.zeros_like(sumsq_ref)
            cnt_ref[...] = jnp.zeros_like(cnt_ref)

        sums_ref[...] += lax.dot_general(
            h, mask, dnl, preferred_element_type=jnp.float32, precision=_HP)
        sumsq_ref[...] += lax.dot_general(
            h * h, mask, dnl, preferred_element_type=jnp.float32,
            precision=_HP)
        cnt_ref[...] += lax.dot_general(
            jnp.ones_like(h), mask, dnl, preferred_element_type=jnp.float32,
            precision=_HP)

    @pl.when(p == 1)
    def _():
        cnt = cnt_ref[...]
        mean = sums_ref[...] / cnt
        ex2 = sumsq_ref[...] / cnt
        mm = mean * gnms_ref[...]
        var = ex2 - 2.0 * mean * mm + mm * mm
        istd = lax.rsqrt(var + _EPS)
        ag = gnw_ref[...] * istd
        bg = gnb_ref[...] - gnw_ref[...] * mm * istd
        dns = (((1,), (0,)), ((), ()))
        an = lax.dot_general(ag, mask, dns,
                             preferred_element_type=jnp.float32,
                             precision=_HP)
        bn_ = lax.dot_general(bg, mask, dns,
                              preferred_element_type=jnp.float32,
                              precision=_HP)
        hnt = an * h + bn_
        hnt_ref[...] = hnt
        hn_ref[...] = hnt.T


def _tc_ab(x1, s0r, s1r, br, w1c, r1c, b1c, gnwc, gnbc, gnmsc):
    grid = (_N + _BN - 1) // _BN
    row = pl.BlockSpec((1, _BN), lambda p, i: (0, i))
    w = pl.BlockSpec((8, 1), lambda p, i: (0, 0))
    st = pl.BlockSpec((8, _G), lambda p, i: (0, 0))
    return pl.pallas_call(
        _ab_body,
        grid=(2, grid),
        in_specs=[row, row, row, row, w, w, w, w, w, w],
        out_specs=[pl.BlockSpec((8, _BN), lambda p, i: (0, i)),
                   pl.BlockSpec((_BN, 8), lambda p, i: (i, 0)),
                   st, st, st],
        out_shape=[jax.ShapeDtypeStruct((8, _N), jnp.float32),
                   jax.ShapeDtypeStruct((_N, 8), jnp.float32)] +
                  [jax.ShapeDtypeStruct((8, _G), jnp.float32)] * 3,
    )(x1, s0r, s1r, br, w1c, r1c, b1c, gnwc, gnbc, gnmsc)


def _final_body(hnt_ref, a0_ref, a1_ref, wmu_ref, rmu_ref, bmu_ref,
                wlv_ref, rlv_ref, blv_ref, mu_ref, lv_ref):
    a = a0_ref[...] + a1_ref[...]
    hnt = hnt_ref[...]
    dn = (((1,), (0,)), ((), ()))
    mu_ref[...] = (lax.dot_general(wmu_ref[...], a, dn,
                                   preferred_element_type=jnp.float32,
                                   precision=_HP)
                   + lax.dot_general(rmu_ref[...], hnt, dn,
                                     preferred_element_type=jnp.float32,
                                     precision=_HP)
                   + bmu_ref[...])
    lv_ref[...] = (lax.dot_general(wlv_ref[...], a, dn,
                                   preferred_element_type=jnp.float32,
                                   precision=_HP)
                   + lax.dot_general(rlv_ref[...], hnt, dn,
                                     preferred_element_type=jnp.float32,
                                     precision=_HP)
                   + blv_ref[...])


def _tc_final(hnt, a0t, a1t, wmut, rmut, bmut, wlvt, rlvt, blvt):
    grid = (_N + _BN - 1) // _BN
    blk8 = pl.BlockSpec((8, _BN), lambda i: (0, i))
    blk16 = pl.BlockSpec((16, _BN), lambda i: (0, i))
    w168 = pl.BlockSpec((16, 8), lambda i: (0, 0))
    w161 = pl.BlockSpec((16, 1), lambda i: (0, 0))
    return pl.pallas_call(
        _final_body,
        grid=(grid,),
        in_specs=[blk8, blk8, blk8, w168, w168, w161, w168, w168, w161],
        out_specs=[blk16, blk16],
        out_shape=[jax.ShapeDtypeStruct((16, _N), jnp.float32)] * 2,
    )(hnt, a0t, a1t, wmut, rmut, bmut, wlvt, rlvt, blvt)


def kernel(x, edge_index, edge_weight, batch, W1, b1, R1,
           gn_w, gn_b, gn_ms, Wmu, bmu, Rmu, Wlv, blv, Rlv):
    e = edge_weight.shape[0]
    nchunks = -(-e // (_NW * _CHUNK))
    nchunks += nchunks % 2
    ep = _NW * _CHUNK * nchunks
    pad = ep - e

    if pad:
        pad_idx = (jnp.arange(pad, dtype=jnp.int32) * 16) % _N
        src2 = jnp.concatenate([edge_index[0], pad_idx])
        dst2 = jnp.concatenate([edge_index[1], pad_idx])
        ew2 = jnp.concatenate([edge_weight, jnp.zeros((pad,), jnp.float32)])
    else:
        src2, dst2, ew2 = edge_index[0], edge_index[1], edge_weight

    xf = x[:, 0]
    s0, s1 = _sc_pass1(xf, src2, dst2, ew2, nchunks)

    hnt, hn, _, _, _ = _tc_ab(
        xf[None, :], s0[None, :], s1[None, :], batch[None, :],
        W1.reshape(8, 1), R1.reshape(8, 1), b1[:, None],
        gn_w[:, None], gn_b[:, None], gn_ms[:, None])

    a0t, a1t = _sc_pass2(hn, src2, dst2, ew2, nchunks)

    mut, lvt = _tc_final(hnt, a0t, a1t,
                         Wmu.T, Rmu.T, bmu[:, None],
                         Wlv.T, Rlv.T, blv[:, None])
    return (mut.T, lvt.T)
